# baseline (speedup 1.0000x reference)
import functools

import jax
import jax.numpy as jnp
from jax import lax
from jax.experimental import pallas as pl
from jax.experimental.pallas import tpu as pltpu
from jax.experimental.pallas import tpu_sc as plsc

S = 50000
D = 128
NC, NS = 2, 16
NW = NC * NS
NB = 2
CMAX = 496
SPAN_A, SPAN_B = 1568, 1480
NWA = 30
SIZES_A = [496] * 3 + [80]
SIZES_B = [496, 496, 488]


def _span_pipeline(base, sizes, x_hbm, sub_hbm, idx_hbm, out_hbm,
                   idx_all, rows, sem_i, sem_s, sem_g, sem_o):
    K = len(sizes)
    offs = [sum(sizes[:j]) for j in range(K)]
    total = sum(sizes)

    def idx_desc():
        return pltpu.make_async_copy(
            idx_hbm.at[pl.ds(0, total)], idx_all.at[pl.ds(0, total)], sem_i)

    def L(j):
        b = j % NB
        pltpu.async_copy(sub_hbm.at[pl.ds(base + offs[j], sizes[j])],
                         rows[b].at[pl.ds(0, sizes[j])], sem_s[b])

    def G(j):
        b = j % NB
        pltpu.make_async_copy(sub_hbm.at[pl.ds(0, sizes[j])],
                              rows[b].at[pl.ds(0, sizes[j])], sem_s[b]).wait()
        pg = -(-sizes[j] // 16) * 16
        for k in range(pg // 16):
            pltpu.async_copy(x_hbm.at[idx_all.at[pl.ds(offs[j] + 16 * k, 16)]],
                             rows[b].at[pl.ds(16 * k, 16)],
                             sem_g[b], add=True)

    def W(j):
        b = j % NB
        pg = -(-sizes[j] // 16) * 16
        pltpu.make_async_copy(x_hbm.at[pl.ds(0, pg)],
                              rows[b].at[pl.ds(0, pg)], sem_g[b]).wait()
        pltpu.async_copy(rows[b].at[pl.ds(0, sizes[j])],
                         out_hbm.at[pl.ds(base + offs[j], sizes[j])], sem_o[b])

    def Dr(j):
        b = j % NB
        pltpu.make_async_copy(rows[b].at[pl.ds(0, sizes[j])],
                              out_hbm.at[pl.ds(0, sizes[j])], sem_o[b]).wait()

    pltpu.async_copy(idx_hbm.at[pl.ds(base, total)],
                     idx_all.at[pl.ds(0, total)], sem_i)
    L(0)
    if K > 1:
        L(1)
    idx_desc().wait()
    if total % 16:
        idx_all[pl.ds(total, 16)] = jnp.zeros((16,), jnp.int32)
    G(0)
    for j in range(K):
        W(j)
        if j - 1 >= 0:
            Dr(j - 1)
        if j + 2 < K:
            L(j + 2)
        if j + 1 < K:
            G(j + 1)
    Dr(K - 1)


def _sc_body(x_hbm, sub_hbm, idx_hbm, out_hbm, idx_all, *sc):
    wid = lax.axis_index("s") * NC + lax.axis_index("c")
    rows = sc[:NB]
    si = sc[NB]
    sem_s = sc[NB + 1:2 * NB + 1]
    sem_g = sc[2 * NB + 1:3 * NB + 1]
    sem_o = sc[3 * NB + 1:4 * NB + 1]
    args = (x_hbm, sub_hbm, idx_hbm, out_hbm,
            idx_all, rows, si, sem_s, sem_g, sem_o)

    @pl.when(wid < NWA)
    def _():
        _span_pipeline(wid * SPAN_A, SIZES_A, *args)

    @pl.when(wid >= NWA)
    def _():
        _span_pipeline(NWA * SPAN_A + (wid - NWA) * SPAN_B, SIZES_B, *args)


def kernel(x, sub_representations, new_nodes):
    idx = new_nodes.reshape(-1)

    mesh = plsc.VectorSubcoreMesh(core_axis_name="c", subcore_axis_name="s")
    run = functools.partial(
        pl.kernel,
        mesh=mesh,
        out_type=jax.ShapeDtypeStruct((S, D), jnp.float32),
        scratch_types=(
            [pltpu.VMEM((SPAN_A,), jnp.int32)]
            + [pltpu.VMEM((CMAX, D), jnp.float32) for _ in range(NB)]
            + [pltpu.SemaphoreType.DMA for _ in range(1 + 3 * NB)]
        ),
    )(_sc_body)
    return run(x, sub_representations, idx)


# speedup vs baseline: 1.0279x; 1.0279x over previous
import functools

import jax
import jax.numpy as jnp
from jax import lax
from jax.experimental import pallas as pl
from jax.experimental.pallas import tpu as pltpu
from jax.experimental.pallas import tpu_sc as plsc

S = 50000
D = 128
NC, NS = 2, 16
NW = NC * NS
NB = 2
CMAX = 504
SPAN_A, SPAN_B = 1568, 1480
NWA = 30
SIZES_A = [504] * 3 + [56]
SIZES_B = [504, 504, 472]


def _span_pipeline(base, sizes, x_hbm, sub_hbm, idx_hbm, out_hbm,
                   idx_all, rows, sem_i, sem_s, sem_g, sem_o):
    K = len(sizes)
    offs = [sum(sizes[:j]) for j in range(K)]
    total = sum(sizes)

    def idx_desc():
        return pltpu.make_async_copy(
            idx_hbm.at[pl.ds(0, total)], idx_all.at[pl.ds(0, total)], sem_i)

    def L(j):
        b = j % NB
        pltpu.async_copy(sub_hbm.at[pl.ds(base + offs[j], sizes[j])],
                         rows[b].at[pl.ds(0, sizes[j])], sem_s[b])

    def G(j):
        b = j % NB
        pltpu.make_async_copy(sub_hbm.at[pl.ds(0, sizes[j])],
                              rows[b].at[pl.ds(0, sizes[j])], sem_s[b]).wait()
        pltpu.async_copy(x_hbm.at[idx_all.at[pl.ds(offs[j], sizes[j])]],
                         rows[b].at[pl.ds(0, sizes[j])], sem_g[b], add=True)

    def W(j):
        b = j % NB
        pltpu.make_async_copy(x_hbm.at[idx_all.at[pl.ds(offs[j], sizes[j])]],
                              rows[b].at[pl.ds(0, sizes[j])], sem_g[b]).wait()
        pltpu.async_copy(rows[b].at[pl.ds(0, sizes[j])],
                         out_hbm.at[pl.ds(base + offs[j], sizes[j])], sem_o[b])

    def Dr(j):
        b = j % NB
        pltpu.make_async_copy(rows[b].at[pl.ds(0, sizes[j])],
                              out_hbm.at[pl.ds(0, sizes[j])], sem_o[b]).wait()

    pltpu.async_copy(idx_hbm.at[pl.ds(base, total)],
                     idx_all.at[pl.ds(0, total)], sem_i)
    L(0)
    if K > 1:
        L(1)
    idx_desc().wait()
    G(0)
    for j in range(K):
        W(j)
        if j - 1 >= 0:
            Dr(j - 1)
        if j + 2 < K:
            L(j + 2)
        if j + 1 < K:
            G(j + 1)
    Dr(K - 1)


def _sc_body(x_hbm, sub_hbm, idx_hbm, out_hbm, idx_all, *sc):
    wid = lax.axis_index("s") * NC + lax.axis_index("c")
    rows = sc[:NB]
    si = sc[NB]
    sem_s = sc[NB + 1:2 * NB + 1]
    sem_g = sc[2 * NB + 1:3 * NB + 1]
    sem_o = sc[3 * NB + 1:4 * NB + 1]
    args = (x_hbm, sub_hbm, idx_hbm, out_hbm,
            idx_all, rows, si, sem_s, sem_g, sem_o)

    @pl.when(wid < NWA)
    def _():
        _span_pipeline(wid * SPAN_A, SIZES_A, *args)

    @pl.when(wid >= NWA)
    def _():
        _span_pipeline(NWA * SPAN_A + (wid - NWA) * SPAN_B, SIZES_B, *args)


def kernel(x, sub_representations, new_nodes):
    idx = new_nodes.reshape(-1)

    mesh = plsc.VectorSubcoreMesh(core_axis_name="c", subcore_axis_name="s")
    run = functools.partial(
        pl.kernel,
        mesh=mesh,
        out_type=jax.ShapeDtypeStruct((S, D), jnp.float32),
        scratch_types=(
            [pltpu.VMEM((SPAN_A,), jnp.int32)]
            + [pltpu.VMEM((CMAX, D), jnp.float32) for _ in range(NB)]
            + [pltpu.SemaphoreType.DMA for _ in range(1 + 3 * NB)]
        ),
    )(_sc_body)
    return run(x, sub_representations, idx)
